# Initial kernel scaffold; baseline (speedup 1.0000x reference)
import jax
import jax.numpy as jnp
from jax.experimental import pallas as pl


def kernel(input_decoder, embedding_table):
    raise NotImplementedError("write your pallas kernel here")


# speedup vs baseline: 1.4344x; 1.4344x over previous
import functools
import math

import jax
import jax.numpy as jnp
from jax import lax
from jax.experimental import pallas as pl
from jax.experimental.pallas import tpu as pltpu
from jax.experimental.pallas import tpu_sc as plsc

EMBED = 32
NUM_CORES = 2
NUM_SUBCORES = 16
NUM_WORKERS = NUM_CORES * NUM_SUBCORES
FACTOR = math.sqrt(float(EMBED))


@functools.partial(jax.jit, static_argnames=("chunk",))
def _embed_lookup(idx_flat, table, chunk=3200):
    b_total = idx_flat.shape[0]
    b_per_w = b_total // NUM_WORKERS
    n_chunks = b_per_w // chunk

    mesh = plsc.VectorSubcoreMesh(core_axis_name="c", subcore_axis_name="s")

    @functools.partial(
        pl.kernel,
        out_type=jax.ShapeDtypeStruct((b_total, EMBED), jnp.float32),
        mesh=mesh,
        scratch_types=[
            pltpu.VMEM((chunk,), jnp.int32),
            pltpu.VMEM((chunk, EMBED), jnp.float32),
            pltpu.SemaphoreType.DMA,
        ],
        compiler_params=pltpu.CompilerParams(use_tc_tiling_on_sc=False),
    )
    def k(idx_hbm, table_hbm, out_hbm, idx_v, rows_v, sem):
        wid = lax.axis_index("s") * NUM_CORES + lax.axis_index("c")
        base = wid * b_per_w

        def chunk_body(ci, carry):
            off = base + ci * chunk
            pltpu.sync_copy(idx_hbm.at[pl.ds(off, chunk)], idx_v)
            pltpu.async_copy(table_hbm.at[idx_v], rows_v, sem).wait()

            def scale_body(r, c):
                rows_v[r, pl.ds(0, 16)] = rows_v[r, pl.ds(0, 16)] * FACTOR
                rows_v[r, pl.ds(16, 16)] = rows_v[r, pl.ds(16, 16)] * FACTOR
                return c

            lax.fori_loop(0, chunk, scale_body, 0, unroll=4)
            pltpu.sync_copy(rows_v, out_hbm.at[pl.ds(off, chunk)])
            return carry

        lax.fori_loop(0, n_chunks, chunk_body, 0)

    return k(idx_flat, table)


def kernel(input_decoder, embedding_table):
    batch, seq = input_decoder.shape
    idx_flat = input_decoder.reshape(-1).astype(jnp.int32)
    out = _embed_lookup(idx_flat, embedding_table)
    return out.reshape(batch, seq, EMBED)
